# baseline (speedup 1.0000x reference)
import functools

import jax
import jax.numpy as jnp
from jax.experimental import pallas as pl

_H = 16
_W = 16
_HW = _H * _W
_DIM = 64
_BS = 128


def _embed_kernel(colors_ref, seen_ref, arm_ref, ang_ref, loc_ref, tgt_ref, out_ref):
    bs = colors_ref.shape[0]
    colors = colors_ref[...]
    colors_t = jnp.transpose(colors, (0, 2, 1))
    out_ref[:, 0:8, :] = colors_t

    out_ref[:, 8:9, :] = seen_ref[...][:, None, :]

    arm = arm_ref[...] / ang_ref[...]
    out_ref[:, 9:13, :] = jnp.broadcast_to(arm[:, :, None], (bs, 4, _HW))

    lanes = jax.lax.broadcasted_iota(jnp.int32, (bs, _HW), 1)
    loc = loc_ref[...]
    loc_idx = loc[:, 0:1] * _W + loc[:, 1:2]
    out_ref[:, 13:14, :] = (lanes == loc_idx).astype(jnp.float32)[:, None, :]
    tgt = tgt_ref[...]
    tgt_idx = tgt[:, 0:1] * _W + tgt[:, 1:2]
    out_ref[:, 14:15, :] = (lanes == tgt_idx).astype(jnp.float32)[:, None, :]

    out_ref[:, 15:_DIM, :] = jnp.zeros((bs, _DIM - 15, _HW), jnp.float32)


@jax.jit
def kernel(colors, seen, arm, angle_sizes, loc, target):
    B = colors.shape[0]
    colors2 = colors.reshape(B, _HW, 8)
    seen2 = seen.reshape(B, _HW)
    ang2 = jnp.broadcast_to(angle_sizes.reshape(1, 4), (_BS, 4))

    grid = (B // _BS,)
    out = pl.pallas_call(
        _embed_kernel,
        grid=grid,
        in_specs=[
            pl.BlockSpec((_BS, _HW, 8), lambda i: (i, 0, 0)),
            pl.BlockSpec((_BS, _HW), lambda i: (i, 0)),
            pl.BlockSpec((_BS, 4), lambda i: (i, 0)),
            pl.BlockSpec((_BS, 4), lambda i: (0, 0)),
            pl.BlockSpec((_BS, 2), lambda i: (i, 0)),
            pl.BlockSpec((_BS, 2), lambda i: (i, 0)),
        ],
        out_specs=pl.BlockSpec((_BS, _DIM, _HW), lambda i: (i, 0, 0)),
        out_shape=jax.ShapeDtypeStruct((B, _DIM, _HW), jnp.float32),
    )(colors2, seen2, arm, ang2, loc, target)
    return out.reshape(B, _DIM, _H, _W)
